# Initial kernel scaffold; baseline (speedup 1.0000x reference)
import jax
import jax.numpy as jnp
from jax.experimental import pallas as pl


def kernel(x, edge_index, edge_attr, We1, be1, W1a, b1a, W1b, b1b, We2, be2, W2a, b2a, W2b, b2b, Wp, bp):
    raise NotImplementedError("write your pallas kernel here")


# speedup vs baseline: 2.4091x; 2.4091x over previous
import functools

import jax
import jax.numpy as jnp
from jax import lax
from jax.experimental import pallas as pl
from jax.experimental.pallas import tpu as pltpu
from jax.experimental.pallas import tpu_sc as plsc

_N = 10000
_E = 320000
_DE = 16
_D = 128
_A = 32

_NUM_TILES = 32
_EPT = _E // _NUM_TILES
_CHUNK = 80
_NCHUNK = _EPT // _CHUNK
_ZC = 200
_NZ = _N // _ZC
_LANES = _D // 16


def _sc_edge_stage(h, e, src, dst):
    mesh = plsc.VectorSubcoreMesh(core_axis_name="c", subcore_axis_name="s")

    @functools.partial(
        pl.kernel,
        mesh=mesh,
        out_type=jax.ShapeDtypeStruct((2, _N, _D), jnp.float32),
        scratch_types=[
            pltpu.VMEM((_CHUNK,), jnp.int32),
            pltpu.VMEM((_CHUNK,), jnp.int32),
            pltpu.VMEM((_CHUNK, _D), jnp.float32),
            pltpu.VMEM((_CHUNK, _D), jnp.float32),
            pltpu.VMEM((_ZC, _D), jnp.float32),
            pltpu.VMEM_SHARED((_N, _D), jnp.float32),
            pltpu.SemaphoreType.DMA,
        ],
    )
    def k(h_hbm, e_hbm, src_hbm, dst_hbm, out_hbm,
          srcv, dstv, rows, ev, zbuf, acc, sem):
        c = lax.axis_index("c")
        s = lax.axis_index("s")
        wid = c * 16 + s

        zero16 = jnp.zeros((16,), jnp.float32)

        def zrow(r, carry):
            for j in range(_LANES):
                zbuf[r, pl.ds(j * 16, 16)] = zero16
            return carry

        lax.fori_loop(0, _ZC, zrow, 0)

        for kk in range((_NZ + 15) // 16):
            idx = s + 16 * kk

            @pl.when(idx < _NZ)
            def _():
                pltpu.sync_copy(zbuf, acc.at[pl.ds(idx * _ZC, _ZC)])

        plsc.subcore_barrier()

        base0 = wid * _EPT

        def chunk(i, carry):
            base = base0 + i * _CHUNK
            pltpu.sync_copy(src_hbm.at[pl.ds(base, _CHUNK)], srcv)
            pltpu.sync_copy(dst_hbm.at[pl.ds(base, _CHUNK)], dstv)
            pltpu.async_copy(h_hbm.at[srcv], rows, sem).wait()
            pltpu.sync_copy(e_hbm.at[pl.ds(base, _CHUNK)], ev)

            def rowbody(r, rcarry):
                for j in range(_LANES):
                    sl = pl.ds(j * 16, 16)
                    rows[r, sl] = jnp.maximum(rows[r, sl] + ev[r, sl], 0.0)
                return rcarry

            lax.fori_loop(0, _CHUNK, rowbody, 0)
            pltpu.sync_copy(rows, acc.at[dstv], add=True)
            return carry

        lax.fori_loop(0, _NCHUNK, chunk, 0)
        plsc.subcore_barrier()

        for kk in range((_NZ + 15) // 16):
            idx = s + 16 * kk

            @pl.when(idx < _NZ)
            def _():
                pltpu.sync_copy(acc.at[pl.ds(idx * _ZC, _ZC)],
                                out_hbm.at[c, pl.ds(idx * _ZC, _ZC)])

    return k(h, e, src, dst)


def _tc_edge_mm(edge_attr, We1, be1, We2, be2):
    be = 2000
    grid = (_E // be,)

    def body(ea_ref, w1_ref, b1_ref, w2_ref, b2_ref, o1_ref, o2_ref):
        ea = ea_ref[...]
        o1_ref[...] = (jnp.dot(ea, w1_ref[...],
                               preferred_element_type=jnp.float32)
                       + b1_ref[...])
        o2_ref[...] = (jnp.dot(ea, w2_ref[...],
                               preferred_element_type=jnp.float32)
                       + b2_ref[...])

    return pl.pallas_call(
        body,
        grid=grid,
        in_specs=[
            pl.BlockSpec((be, _DE), lambda i: (i, 0)),
            pl.BlockSpec((_DE, _D), lambda i: (0, 0)),
            pl.BlockSpec((1, _D), lambda i: (0, 0)),
            pl.BlockSpec((_DE, _D), lambda i: (0, 0)),
            pl.BlockSpec((1, _D), lambda i: (0, 0)),
        ],
        out_specs=[
            pl.BlockSpec((be, _D), lambda i: (i, 0)),
            pl.BlockSpec((be, _D), lambda i: (i, 0)),
        ],
        out_shape=[jax.ShapeDtypeStruct((_E, _D), jnp.float32)] * 2,
    )(edge_attr, We1, be1.reshape(1, _D), We2, be2.reshape(1, _D))


def _tc_node_mlp(h, parts, Wa, ba, Wb, bb, Wp=None, bp=None):
    bn = 2000
    grid = (_N // bn,)
    project = Wp is not None

    def body(h_ref, p_ref, wa_ref, ba_ref, wb_ref, bb_ref, *rest):
        o_ref = rest[-1]
        z = h_ref[...] + p_ref[0] + p_ref[1]
        z = jnp.maximum(
            jnp.dot(z, wa_ref[...], preferred_element_type=jnp.float32)
            + ba_ref[...], 0.0)
        z = jnp.maximum(
            jnp.dot(z, wb_ref[...], preferred_element_type=jnp.float32)
            + bb_ref[...], 0.0)
        if project:
            wp_ref, bp_ref = rest[0], rest[1]
            o_ref[...] = (jnp.dot(z, wp_ref[...],
                                  preferred_element_type=jnp.float32)
                          + bp_ref[...])
        else:
            o_ref[...] = z

    in_specs = [
        pl.BlockSpec((bn, _D), lambda i: (i, 0)),
        pl.BlockSpec((2, bn, _D), lambda i: (0, i, 0)),
        pl.BlockSpec((_D, _D), lambda i: (0, 0)),
        pl.BlockSpec((1, _D), lambda i: (0, 0)),
        pl.BlockSpec((_D, _D), lambda i: (0, 0)),
        pl.BlockSpec((1, _D), lambda i: (0, 0)),
    ]
    args = [h, parts, Wa, ba.reshape(1, _D), Wb, bb.reshape(1, _D)]
    if project:
        in_specs += [
            pl.BlockSpec((_D, _A), lambda i: (0, 0)),
            pl.BlockSpec((1, _A), lambda i: (0, 0)),
        ]
        args += [Wp, bp.reshape(1, _A)]
        out_dim = _A
    else:
        out_dim = _D

    return pl.pallas_call(
        body,
        grid=grid,
        in_specs=in_specs,
        out_specs=pl.BlockSpec((bn, out_dim), lambda i: (i, 0)),
        out_shape=jax.ShapeDtypeStruct((_N, out_dim), jnp.float32),
    )(*args)


def kernel(x, edge_index, edge_attr, We1, be1, W1a, b1a, W1b, b1b,
           We2, be2, W2a, b2a, W2b, b2b, Wp, bp):
    src = edge_index[0]
    dst = edge_index[1]
    e1, e2 = _tc_edge_mm(edge_attr, We1, be1, We2, be2)
    p1 = _sc_edge_stage(x, e1, src, dst)
    z1 = _tc_node_mlp(x, p1, W1a, b1a, W1b, b1b)
    p2 = _sc_edge_stage(z1, e2, src, dst)
    return _tc_node_mlp(z1, p2, W2a, b2a, W2b, b2b, Wp, bp)


# speedup vs baseline: 4.5319x; 1.8812x over previous
import functools

import jax
import jax.numpy as jnp
from jax import lax
from jax.experimental import pallas as pl
from jax.experimental.pallas import tpu as pltpu
from jax.experimental.pallas import tpu_sc as plsc

_N = 10000
_E = 320000
_DE = 16
_D = 128
_A = 32

_NUM_TILES = 32
_EPT = _E // _NUM_TILES
_CHUNK = 40
_NCHUNK = _EPT // _CHUNK
_ZC = 40
_NZ = _N // _ZC
_LANES = _D // 16
_QD = 8


def _sc_edge_stage(h, e, src3, dst3):
    mesh = plsc.VectorSubcoreMesh(core_axis_name="c", subcore_axis_name="s")

    @functools.partial(
        pl.kernel,
        mesh=mesh,
        out_type=jax.ShapeDtypeStruct((2, _N, _D), jnp.float32),
        scratch_types=[
            pltpu.VMEM((_QD, _CHUNK), jnp.int32),
            pltpu.VMEM((_QD, _CHUNK), jnp.int32),
            pltpu.VMEM((2, _CHUNK, _D), jnp.float32),
            pltpu.VMEM((2, _CHUNK, _D), jnp.float32),
            pltpu.VMEM((2, _CHUNK, _D), jnp.float32),
            pltpu.VMEM_SHARED((_N, _D), jnp.float32),
            pltpu.SemaphoreType.DMA((_QD,)),
            pltpu.SemaphoreType.DMA((_QD,)),
            pltpu.SemaphoreType.DMA((2,)),
            pltpu.SemaphoreType.DMA((2,)),
            pltpu.SemaphoreType.DMA((2,)),
        ],
    )
    def k(h_hbm, e_hbm, src_hbm, dst_hbm, out_hbm,
          srcq, dstq, rows, ev, mbuf, acc, sisem, disem, gsem, esem, ssem):
        c = lax.axis_index("c")
        s = lax.axis_index("s")
        wid = c * 16 + s

        zero16 = jnp.zeros((16,), jnp.float32)

        def zrow(r, carry):
            for j in range(_LANES):
                mbuf[0, r, pl.ds(j * 16, 16)] = zero16
            return carry

        lax.fori_loop(0, _ZC, zrow, 0)

        for kk in range((_NZ + 15) // 16):
            idx = s + 16 * kk

            @pl.when(idx < _NZ)
            def _():
                pltpu.async_copy(mbuf.at[0], acc.at[pl.ds(idx * _ZC, _ZC)],
                                 gsem.at[0])

        for kk in range((_NZ + 15) // 16):
            idx = s + 16 * kk

            @pl.when(idx < _NZ)
            def _():
                pltpu.make_async_copy(mbuf.at[0],
                                      acc.at[pl.ds(idx * _ZC, _ZC)],
                                      gsem.at[0]).wait()

        plsc.subcore_barrier()

        base0 = wid * _EPT

        def idx_fetch(j):
            q = j & (_QD - 1)
            pltpu.async_copy(src_hbm.at[wid, j], srcq.at[q], sisem.at[q])
            pltpu.async_copy(dst_hbm.at[wid, j], dstq.at[q], disem.at[q])

        def start_fetch(i, b):
            q = i & (_QD - 1)
            pltpu.make_async_copy(src_hbm.at[wid, i], srcq.at[q],
                                  sisem.at[q]).wait()
            pltpu.make_async_copy(dst_hbm.at[wid, i], dstq.at[q],
                                  disem.at[q]).wait()
            pltpu.async_copy(h_hbm.at[srcq.at[q]], rows.at[b], gsem.at[b])
            pltpu.async_copy(e_hbm.at[pl.ds(base0 + i * _CHUNK, _CHUNK)],
                             ev.at[b], esem.at[b])

        def consume(i, b):
            q = i & (_QD - 1)
            pltpu.make_async_copy(h_hbm.at[srcq.at[q]], rows.at[b],
                                  gsem.at[b]).wait()
            pltpu.make_async_copy(e_hbm.at[pl.ds(base0 + i * _CHUNK, _CHUNK)],
                                  ev.at[b], esem.at[b]).wait()

            @pl.when(i >= 2)
            def _():
                qp = (i - 2) & (_QD - 1)
                pltpu.make_async_copy(mbuf.at[b], acc.at[dstq.at[qp]],
                                      ssem.at[b]).wait()

            @pl.when(i + 6 < _NCHUNK)
            def _():
                idx_fetch(i + 6)

            def rowbody(r, rcarry):
                for j in range(_LANES):
                    sl = pl.ds(j * 16, 16)
                    mbuf[b, r, sl] = jnp.maximum(
                        rows[b, r, sl] + ev[b, r, sl], 0.0)
                return rcarry

            lax.fori_loop(0, _CHUNK, rowbody, 0)
            pltpu.async_copy(mbuf.at[b], acc.at[dstq.at[q]], ssem.at[b],
                             add=True)

        for j in range(6):
            idx_fetch(j)
        start_fetch(0, 0)
        start_fetch(1, 1)

        def pair(i2, carry):
            p = 2 * i2
            for b in range(2):
                i = p + b

                @pl.when(i < _NCHUNK)
                def _():
                    consume(i, b)

                @pl.when(i + 2 < _NCHUNK)
                def _():
                    start_fetch(i + 2, b)
            return carry

        lax.fori_loop(0, (_NCHUNK + 1) // 2, pair, 0)

        last0 = _NCHUNK - 1 if (_NCHUNK - 1) % 2 == 0 else _NCHUNK - 2
        last1 = _NCHUNK - 1 if (_NCHUNK - 1) % 2 == 1 else _NCHUNK - 2
        pltpu.make_async_copy(mbuf.at[0], acc.at[dstq.at[last0 & (_QD - 1)]],
                              ssem.at[0]).wait()
        pltpu.make_async_copy(mbuf.at[1], acc.at[dstq.at[last1 & (_QD - 1)]],
                              ssem.at[1]).wait()
        plsc.subcore_barrier()

        for kk in range((_NZ + 15) // 16):
            idx = s + 16 * kk

            @pl.when(idx < _NZ)
            def _():
                pltpu.async_copy(acc.at[pl.ds(idx * _ZC, _ZC)],
                                 out_hbm.at[c, pl.ds(idx * _ZC, _ZC)],
                                 gsem.at[0])

        for kk in range((_NZ + 15) // 16):
            idx = s + 16 * kk

            @pl.when(idx < _NZ)
            def _():
                pltpu.make_async_copy(acc.at[pl.ds(idx * _ZC, _ZC)],
                                      out_hbm.at[c, pl.ds(idx * _ZC, _ZC)],
                                      gsem.at[0]).wait()

    return k(h, e, src3, dst3)


def _tc_edge_mm(edge_attr, We1, be1, We2, be2):
    be = 2000
    grid = (_E // be,)

    def body(ea_ref, w1_ref, b1_ref, w2_ref, b2_ref, o1_ref, o2_ref):
        ea = ea_ref[...]
        o1_ref[...] = (jnp.dot(ea, w1_ref[...],
                               preferred_element_type=jnp.float32)
                       + b1_ref[...])
        o2_ref[...] = (jnp.dot(ea, w2_ref[...],
                               preferred_element_type=jnp.float32)
                       + b2_ref[...])

    return pl.pallas_call(
        body,
        grid=grid,
        in_specs=[
            pl.BlockSpec((be, _DE), lambda i: (i, 0)),
            pl.BlockSpec((_DE, _D), lambda i: (0, 0)),
            pl.BlockSpec((1, _D), lambda i: (0, 0)),
            pl.BlockSpec((_DE, _D), lambda i: (0, 0)),
            pl.BlockSpec((1, _D), lambda i: (0, 0)),
        ],
        out_specs=[
            pl.BlockSpec((be, _D), lambda i: (i, 0)),
            pl.BlockSpec((be, _D), lambda i: (i, 0)),
        ],
        out_shape=[jax.ShapeDtypeStruct((_E, _D), jnp.float32)] * 2,
    )(edge_attr, We1, be1.reshape(1, _D), We2, be2.reshape(1, _D))


def _tc_node_mlp(h, parts, Wa, ba, Wb, bb, Wp=None, bp=None):
    bn = 2000
    grid = (_N // bn,)
    project = Wp is not None

    def body(h_ref, p_ref, wa_ref, ba_ref, wb_ref, bb_ref, *rest):
        o_ref = rest[-1]
        z = h_ref[...] + p_ref[0] + p_ref[1]
        z = jnp.maximum(
            jnp.dot(z, wa_ref[...], preferred_element_type=jnp.float32)
            + ba_ref[...], 0.0)
        z = jnp.maximum(
            jnp.dot(z, wb_ref[...], preferred_element_type=jnp.float32)
            + bb_ref[...], 0.0)
        if project:
            wp_ref, bp_ref = rest[0], rest[1]
            o_ref[...] = (jnp.dot(z, wp_ref[...],
                                  preferred_element_type=jnp.float32)
                          + bp_ref[...])
        else:
            o_ref[...] = z

    in_specs = [
        pl.BlockSpec((bn, _D), lambda i: (i, 0)),
        pl.BlockSpec((2, bn, _D), lambda i: (0, i, 0)),
        pl.BlockSpec((_D, _D), lambda i: (0, 0)),
        pl.BlockSpec((1, _D), lambda i: (0, 0)),
        pl.BlockSpec((_D, _D), lambda i: (0, 0)),
        pl.BlockSpec((1, _D), lambda i: (0, 0)),
    ]
    args = [h, parts, Wa, ba.reshape(1, _D), Wb, bb.reshape(1, _D)]
    if project:
        in_specs += [
            pl.BlockSpec((_D, _A), lambda i: (0, 0)),
            pl.BlockSpec((1, _A), lambda i: (0, 0)),
        ]
        args += [Wp, bp.reshape(1, _A)]
        out_dim = _A
    else:
        out_dim = _D

    return pl.pallas_call(
        body,
        grid=grid,
        in_specs=in_specs,
        out_specs=pl.BlockSpec((bn, out_dim), lambda i: (i, 0)),
        out_shape=jax.ShapeDtypeStruct((_N, out_dim), jnp.float32),
    )(*args)


def kernel(x, edge_index, edge_attr, We1, be1, W1a, b1a, W1b, b1b,
           We2, be2, W2a, b2a, W2b, b2b, Wp, bp):
    src3 = edge_index[0].reshape(_NUM_TILES, _NCHUNK, _CHUNK)
    dst3 = edge_index[1].reshape(_NUM_TILES, _NCHUNK, _CHUNK)
    e1, e2 = _tc_edge_mm(edge_attr, We1, be1, We2, be2)
    p1 = _sc_edge_stage(x, e1, src3, dst3)
    z1 = _tc_node_mlp(x, p1, W1a, b1a, W1b, b1b)
    p2 = _sc_edge_stage(z1, e2, src3, dst3)
    return _tc_node_mlp(z1, p2, W2a, b2a, W2b, b2b, Wp, bp)
